# Initial kernel scaffold; baseline (speedup 1.0000x reference)
import jax
import jax.numpy as jnp
from jax.experimental import pallas as pl


def kernel(inputs, global_gamma, global_beta, domain_gamma, domain_beta, domain_index):
    raise NotImplementedError("write your pallas kernel here")


# speedup vs baseline: 2.7772x; 2.7772x over previous
import functools

import jax
import jax.numpy as jnp
from jax.experimental import pallas as pl
from jax.experimental.pallas import tpu as pltpu

NUM_DOMAIN = 8
BATCH = 4096
DIM = 512
EPS = 1e-3
BB = 512
NB = BATCH // BB


def _stats_body(x_ref, di_ref, gg_ref, gb_ref, dg_ref, db_ref,
                scale_ref, bias_ref, acc_sum, acc_ss, acc_cnt):
    i = pl.program_id(0)

    @pl.when(i == 0)
    def _():
        acc_sum[...] = jnp.zeros_like(acc_sum)
        acc_ss[...] = jnp.zeros_like(acc_ss)
        acc_cnt[...] = jnp.zeros_like(acc_cnt)

    x = x_ref[...]
    di = di_ref[0, 0, :]
    oh = (jax.lax.broadcasted_iota(jnp.int32, (NUM_DOMAIN, BB), 0)
          == di[None, :]).astype(jnp.float32)
    acc_sum[...] += jnp.dot(oh, x, preferred_element_type=jnp.float32)
    acc_ss[...] += jnp.dot(oh, x * x, preferred_element_type=jnp.float32)
    acc_cnt[...] += jnp.sum(oh, axis=1, keepdims=True)

    @pl.when(i == NB - 1)
    def _():
        cnt = jnp.maximum(acc_cnt[...], 1.0)
        mean = acc_sum[...] / cnt
        var = acc_ss[...] / cnt - mean * mean
        rstd = jax.lax.rsqrt(var + EPS)
        g = gg_ref[...] + dg_ref[...]
        scale = g * rstd
        scale_ref[...] = scale
        bias_ref[...] = (gb_ref[...] + db_ref[...]) - mean * scale


def _apply_body(x_ref, di_ref, scale_ref, bias_ref, out_ref):
    x = x_ref[...]
    di = di_ref[0, 0, :]
    oh = (di[:, None]
          == jax.lax.broadcasted_iota(jnp.int32, (BB, NUM_DOMAIN), 1)
          ).astype(jnp.float32)
    s = jnp.dot(oh, scale_ref[...], preferred_element_type=jnp.float32)
    b = jnp.dot(oh, bias_ref[...], preferred_element_type=jnp.float32)
    out_ref[...] = x * s + b


@functools.partial(jax.jit, static_argnames=("interpret",))
def kernel(inputs, global_gamma, global_beta, domain_gamma, domain_beta,
           domain_index, interpret=False):
    di = domain_index.astype(jnp.int32).reshape(NB, 1, BB)
    gg = global_gamma.reshape(1, DIM)
    gb = global_beta.reshape(1, DIM)

    f32 = jnp.float32
    scale, bias = pl.pallas_call(
        _stats_body,
        grid=(NB,),
        in_specs=[
            pl.BlockSpec((BB, DIM), lambda i: (i, 0)),
            pl.BlockSpec((1, 1, BB), lambda i: (i, 0, 0)),
            pl.BlockSpec((1, DIM), lambda i: (0, 0)),
            pl.BlockSpec((1, DIM), lambda i: (0, 0)),
            pl.BlockSpec((NUM_DOMAIN, DIM), lambda i: (0, 0)),
            pl.BlockSpec((NUM_DOMAIN, DIM), lambda i: (0, 0)),
        ],
        out_specs=[
            pl.BlockSpec((NUM_DOMAIN, DIM), lambda i: (0, 0)),
            pl.BlockSpec((NUM_DOMAIN, DIM), lambda i: (0, 0)),
        ],
        out_shape=[
            jax.ShapeDtypeStruct((NUM_DOMAIN, DIM), f32),
            jax.ShapeDtypeStruct((NUM_DOMAIN, DIM), f32),
        ],
        scratch_shapes=[
            pltpu.VMEM((NUM_DOMAIN, DIM), f32),
            pltpu.VMEM((NUM_DOMAIN, DIM), f32),
            pltpu.VMEM((NUM_DOMAIN, 1), f32),
        ],
        interpret=interpret,
    )(inputs, di, gg, gb, domain_gamma, domain_beta)

    out = pl.pallas_call(
        _apply_body,
        grid=(NB,),
        in_specs=[
            pl.BlockSpec((BB, DIM), lambda i: (i, 0)),
            pl.BlockSpec((1, 1, BB), lambda i: (i, 0, 0)),
            pl.BlockSpec((NUM_DOMAIN, DIM), lambda i: (0, 0)),
            pl.BlockSpec((NUM_DOMAIN, DIM), lambda i: (0, 0)),
        ],
        out_specs=pl.BlockSpec((BB, DIM), lambda i: (i, 0)),
        out_shape=jax.ShapeDtypeStruct((BATCH, DIM), f32),
        interpret=interpret,
    )(inputs, di, scale, bias)
    return out
